# baseline (speedup 1.0000x reference)
from functools import partial

import jax
import jax.numpy as jnp
import numpy as np
from jax import lax
from jax.experimental import pallas as pl
from jax.experimental.pallas import tpu as pltpu
from jax.experimental.pallas import tpu_sc as plsc

N = 10000
NP = 10240
D_IN = 128
H1 = 8
C1 = 8
F1 = H1 * C1
OUT = 40

E_RAW = 320000
E_TOT = E_RAW + N
BLK = 128
NWORK = 32
BPW = 82
NBLK = NWORK * BPW
NBLK_ID = NBLK + 16
EP = NBLK * BLK
ROWS_PER_SUB = NP // 16

W_A1, W_A2, W_D = 80, 64, 16

_IDX_C = np.arange(F1).reshape(H1, C1).T.ravel()
_PERM = np.eye(F1, dtype=np.float32)[_IDX_C].T
_BLKDIAG_MASK = np.kron(np.eye(H1, dtype=np.float32), np.ones((C1, 1), np.float32))


def _tc_proj1(x_ref, w_ref, p_ref, asr_ref, adr_ref, a_out, d_out):
    h = jnp.dot(x_ref[...], w_ref[...], preferred_element_type=jnp.float32)
    hc = jnp.dot(h, p_ref[...], preferred_element_type=jnp.float32)
    s = jnp.dot(h, asr_ref[...], preferred_element_type=jnp.float32)
    d = jnp.dot(h, adr_ref[...], preferred_element_type=jnp.float32)
    a_out[:, 0:F1] = hc
    a_out[:, F1:F1 + H1] = s
    a_out[:, F1 + H1:W_A1] = s
    d_out[:, 0:H1] = d
    d_out[:, H1:W_D] = d


def _tc_mid(p_ref, w2_ref, as2_ref, ad2_ref, b1_ref, a_out, s_out, d_out):
    acc = p_ref[0] + p_ref[1]
    den = acc[:, F1:F1 + H1]
    div = jnp.tile(den, (1, C1)) + 1e-16
    oc = acc[:, 0:F1] / div + b1_ref[...]
    oc = jnp.maximum(oc, 0.0)
    h2 = jnp.dot(oc, w2_ref[...], preferred_element_type=jnp.float32)
    s2 = jnp.dot(h2, as2_ref[...], preferred_element_type=jnp.float32)
    d2 = jnp.dot(h2, ad2_ref[...], preferred_element_type=jnp.float32)
    rows = h2.shape[0]
    a_out[:, 0:OUT] = h2
    a_out[:, OUT:48] = jnp.zeros((rows, 48 - OUT), jnp.float32)
    s_out[...] = s2
    d_out[...] = d2


def _tc_out(p_ref, b2_ref, o_ref):
    acc = p_ref[0] + p_ref[1]
    den = acc[:, 48:49] + 1e-16
    z = acc[:, 0:OUT] / den + b2_ref[...]
    m = jnp.max(z, axis=1, keepdims=True)
    ez = jnp.exp(z - m)
    lse = jnp.log(jnp.sum(ez, axis=1, keepdims=True))
    o_ref[...] = z - m - lse


ROWB = 1280


def _proj1_call(xp, W1, P, Asrc, Adst):
    return pl.pallas_call(
        _tc_proj1,
        grid=(NP // ROWB,),
        in_specs=[
            pl.BlockSpec((ROWB, D_IN), lambda i: (i, 0)),
            pl.BlockSpec((D_IN, F1), lambda i: (0, 0)),
            pl.BlockSpec((F1, F1), lambda i: (0, 0)),
            pl.BlockSpec((F1, H1), lambda i: (0, 0)),
            pl.BlockSpec((F1, H1), lambda i: (0, 0)),
        ],
        out_specs=[
            pl.BlockSpec((ROWB, W_A1), lambda i: (i, 0)),
            pl.BlockSpec((ROWB, W_D), lambda i: (i, 0)),
        ],
        out_shape=[
            jax.ShapeDtypeStruct((NP, W_A1), jnp.float32),
            jax.ShapeDtypeStruct((NP, W_D), jnp.float32),
        ],
    )(xp, W1, P, Asrc, Adst)


def _mid_call(parts1, W2q, as2c, ad2c, b1c):
    return pl.pallas_call(
        _tc_mid,
        grid=(NP // ROWB,),
        in_specs=[
            pl.BlockSpec((2, ROWB, W_A1), lambda i: (0, i, 0)),
            pl.BlockSpec((F1, OUT), lambda i: (0, 0)),
            pl.BlockSpec((OUT, 1), lambda i: (0, 0)),
            pl.BlockSpec((OUT, 1), lambda i: (0, 0)),
            pl.BlockSpec((1, F1), lambda i: (0, 0)),
        ],
        out_specs=[
            pl.BlockSpec((ROWB, 48), lambda i: (i, 0)),
            pl.BlockSpec((ROWB, 1), lambda i: (i, 0)),
            pl.BlockSpec((ROWB, 1), lambda i: (i, 0)),
        ],
        out_shape=[
            jax.ShapeDtypeStruct((NP, 48), jnp.float32),
            jax.ShapeDtypeStruct((NP, 1), jnp.float32),
            jax.ShapeDtypeStruct((NP, 1), jnp.float32),
        ],
    )(parts1, W2q, as2c, ad2c, b1c)


def _out_call(parts2, b2r):
    return pl.pallas_call(
        _tc_out,
        grid=(NP // ROWB,),
        in_specs=[
            pl.BlockSpec((2, ROWB, W_A2), lambda i: (0, i, 0)),
            pl.BlockSpec((1, OUT), lambda i: (0, 0)),
        ],
        out_specs=pl.BlockSpec((ROWB, OUT), lambda i: (i, 0)),
        out_shape=jax.ShapeDtypeStruct((NP, OUT), jnp.float32),
    )(parts2, b2r)


def _edge_pass(width, msg_regs, a_tab, d_tab, sidx2d, didx2d, zeros_tab):
    @partial(
        pl.kernel,
        out_type=jax.ShapeDtypeStruct((2, NP, width), jnp.float32),
        mesh=plsc.VectorSubcoreMesh(core_axis_name="c", subcore_axis_name="s"),
        scratch_types=[
            pltpu.VMEM((BPW + 7, BLK), jnp.int32),
            pltpu.VMEM((BPW + 7, BLK), jnp.int32),
            pltpu.VMEM((BLK, width), jnp.float32),
            pltpu.VMEM((BLK, width), jnp.float32),
            pltpu.VMEM((BLK, W_D), jnp.float32),
            pltpu.VMEM((BLK, W_D), jnp.float32),
            pltpu.VMEM((BLK, width), jnp.float32),
            pltpu.VMEM((BLK, width), jnp.float32),
            pltpu.VMEM_SHARED((NP, width), jnp.float32),
            pltpu.SemaphoreType.DMA,
            pltpu.SemaphoreType.DMA,
            pltpu.SemaphoreType.DMA,
            pltpu.SemaphoreType.DMA,
            pltpu.SemaphoreType.DMA,
            pltpu.SemaphoreType.DMA,
        ],
        compiler_params=pltpu.CompilerParams(use_tc_tiling_on_sc=False),
    )
    def kern(a_hbm, d_hbm, s_hbm, di_hbm, z_hbm, out_hbm,
             sidx, didx, bufa0, bufa1, bufd0, bufd1, bufo0, bufo1, acc,
             semA0, semA1, semD0, semD1, semS0, semS1):
        c = lax.axis_index("c")
        s = lax.axis_index("s")
        w = c * 16 + s
        bufa = (bufa0, bufa1)
        bufd = (bufd0, bufd1)
        bufo = (bufo0, bufo1)
        semA = (semA0, semA1)
        semD = (semD0, semD1)
        semS = (semS0, semS1)
        pltpu.sync_copy(z_hbm.at[pl.ds(s * ROWS_PER_SUB, ROWS_PER_SUB)],
                        acc.at[pl.ds(s * ROWS_PER_SUB, ROWS_PER_SUB)])
        base = (w * BPW) // 8 * 8
        off = w * BPW - base
        pltpu.sync_copy(s_hbm.at[pl.ds(base, BPW + 7)], sidx)
        pltpu.sync_copy(di_hbm.at[pl.ds(base, BPW + 7)], didx)
        plsc.subcore_barrier()

        def gather_start(jj, b):
            pltpu.make_async_copy(a_hbm.at[sidx.at[off + jj]], bufa[b],
                                  semA[b]).start()
            pltpu.make_async_copy(d_hbm.at[didx.at[off + jj]], bufd[b],
                                  semD[b]).start()

        def gather_wait(jj, b):
            pltpu.make_async_copy(a_hbm.at[sidx.at[off + jj]], bufa[b],
                                  semA[b]).wait()
            pltpu.make_async_copy(d_hbm.at[didx.at[off + jj]], bufd[b],
                                  semD[b]).wait()

        def scatter_wait(jj, b):
            pltpu.make_async_copy(bufo[b], acc.at[didx.at[off + jj]],
                                  semS[b]).wait()

        gather_start(0, 0)

        def body(j, b):
            jj = j + b

            @pl.when(jj + 1 < BPW)
            def _():
                gather_start(jj + 1, 1 - b)

            gather_wait(jj, b)

            @pl.when(jj >= 2)
            def _():
                scatter_wait(jj - 2, b)

            @plsc.parallel_loop(0, BLK, 1, unroll=4)
            def _(r):
                sv = bufa[b][r, pl.ds(width - 16, 16)] + bufd[b][r, pl.ds(0, 16)]
                e = jnp.maximum(sv, 0.0) + 0.2 * jnp.minimum(sv, 0.0)
                ee = jnp.exp(e)
                for k in range(msg_regs):
                    bufo[b][r, pl.ds(16 * k, 16)] = (
                        bufa[b][r, pl.ds(16 * k, 16)] * ee)
                bufo[b][r, pl.ds(width - 16, 16)] = ee

            pltpu.make_async_copy(bufo[b], acc.at[didx.at[off + jj]],
                                  semS[b]).start(add=True)

        @pl.loop(0, BPW, step=2)
        def _(j):
            body(j, 0)
            body(j, 1)

        scatter_wait(BPW - 2, 0)
        scatter_wait(BPW - 1, 1)
        plsc.subcore_barrier()
        pltpu.sync_copy(acc.at[pl.ds(s * ROWS_PER_SUB, ROWS_PER_SUB)],
                        out_hbm.at[c].at[pl.ds(s * ROWS_PER_SUB, ROWS_PER_SUB)])

    return kern(a_tab, d_tab, sidx2d, didx2d, zeros_tab)


def _lane_bcast(vec, r):
    idx = jnp.full((16, 1), r, jnp.int32)
    dn = lax.GatherDimensionNumbers(offset_dims=(), collapsed_slice_dims=(0,),
                                    start_index_map=(0,))
    return lax.gather(vec, idx, dn, (1,),
                      mode=lax.GatherScatterMode.PROMISE_IN_BOUNDS)


def _edge_pass2(a_tab, asv_hbm, adv_hbm, sidx2d, didx2d, zeros_tab):
    width = W_A2

    @partial(
        pl.kernel,
        out_type=jax.ShapeDtypeStruct((2, NP, width), jnp.float32),
        mesh=plsc.VectorSubcoreMesh(core_axis_name="c", subcore_axis_name="s"),
        scratch_types=[
            pltpu.VMEM((BPW + 7, BLK), jnp.int32),
            pltpu.VMEM((BPW + 7, BLK), jnp.int32),
            pltpu.VMEM((BLK, 48), jnp.float32),
            pltpu.VMEM((BLK, 48), jnp.float32),
            pltpu.VMEM((BLK, width), jnp.float32),
            pltpu.VMEM((BLK, width), jnp.float32),
            pltpu.VMEM((NP,), jnp.float32),
            pltpu.VMEM((NP,), jnp.float32),
            pltpu.VMEM((16,), jnp.float32),
            pltpu.VMEM_SHARED((NP, width), jnp.float32),
            pltpu.SemaphoreType.DMA,
            pltpu.SemaphoreType.DMA,
            pltpu.SemaphoreType.DMA,
            pltpu.SemaphoreType.DMA,
        ],
        compiler_params=pltpu.CompilerParams(use_tc_tiling_on_sc=False,
                                             needs_layout_passes=False),
    )
    def kern(a_hbm, as_hbm, ad_hbm, s_hbm, di_hbm, z_hbm, out_hbm,
             sidx, didx, bufa0, bufa1, bufo0, bufo1, asv, adv, ebuf, acc,
             semA0, semA1, semS0, semS1):
        c = lax.axis_index("c")
        s = lax.axis_index("s")
        w = c * 16 + s
        bufa = (bufa0, bufa1)
        bufo = (bufo0, bufo1)
        semA = (semA0, semA1)
        semS = (semS0, semS1)
        pltpu.sync_copy(z_hbm.at[pl.ds(s * ROWS_PER_SUB, ROWS_PER_SUB)],
                        acc.at[pl.ds(s * ROWS_PER_SUB, ROWS_PER_SUB)])
        base = (w * BPW) // 8 * 8
        off = w * BPW - base
        pltpu.sync_copy(s_hbm.at[pl.ds(base, BPW + 7)], sidx)
        pltpu.sync_copy(di_hbm.at[pl.ds(base, BPW + 7)], didx)
        pltpu.sync_copy(as_hbm, asv)
        pltpu.sync_copy(ad_hbm, adv)
        plsc.subcore_barrier()

        def gather_start(jj, b):
            pltpu.make_async_copy(a_hbm.at[sidx.at[off + jj]], bufa[b],
                                  semA[b]).start()

        def gather_wait(jj, b):
            pltpu.make_async_copy(a_hbm.at[sidx.at[off + jj]], bufa[b],
                                  semA[b]).wait()

        def scatter_wait(jj, b):
            pltpu.make_async_copy(bufo[b], acc.at[didx.at[off + jj]],
                                  semS[b]).wait()

        gather_start(0, 0)

        def body(j, b):
            jj = j + b

            @pl.when(jj + 1 < BPW)
            def _():
                gather_start(jj + 1, 1 - b)

            gather_wait(jj, b)

            @pl.when(jj >= 2)
            def _():
                scatter_wait(jj - 2, b)

            @pl.loop(0, BLK // 16)
            def _(g):
                g16 = pl.multiple_of(g * 16, 16)
                srcv = sidx[off + jj, pl.ds(g16, 16)]
                dstv = didx[off + jj, pl.ds(g16, 16)]
                sv = (plsc.load_gather(asv, [srcv])
                      + plsc.load_gather(adv, [dstv]))
                e = jnp.maximum(sv, 0.0) + 0.2 * jnp.minimum(sv, 0.0)
                ee = jnp.exp(e)
                for r in range(16):
                    row = g16 + r
                    bc = _lane_bcast(ee, r)
                    for k in range(3):
                        bufo[b][row, pl.ds(16 * k, 16)] = (
                            bufa[b][row, pl.ds(16 * k, 16)] * bc)
                    bufo[b][row, pl.ds(48, 16)] = bc

            pltpu.make_async_copy(bufo[b], acc.at[didx.at[off + jj]],
                                  semS[b]).start(add=True)

        @pl.loop(0, BPW, step=2)
        def _(j):
            body(j, 0)
            body(j, 1)

        scatter_wait(BPW - 2, 0)
        scatter_wait(BPW - 1, 1)
        plsc.subcore_barrier()
        pltpu.sync_copy(acc.at[pl.ds(s * ROWS_PER_SUB, ROWS_PER_SUB)],
                        out_hbm.at[c].at[pl.ds(s * ROWS_PER_SUB, ROWS_PER_SUB)])

    return kern(a_tab, asv_hbm, adv_hbm, sidx2d, didx2d, zeros_tab)


@jax.jit
def kernel(x, edge_index, W1, a_src1, a_dst1, b1, W2, a_src2, a_dst2, b2):
    f32 = jnp.float32
    xp = jnp.concatenate([x, jnp.zeros((NP - N, D_IN), f32)], axis=0)

    P = jnp.asarray(_PERM)
    mask = jnp.asarray(_BLKDIAG_MASK)
    Asrc = mask * a_src1.reshape(-1, 1)
    Adst = mask * a_dst1.reshape(-1, 1)
    W2q = W2[_IDX_C, :]
    b1c = b1[_IDX_C].reshape(1, F1)
    as2c = a_src2.reshape(OUT, 1)
    ad2c = a_dst2.reshape(OUT, 1)
    b2r = b2.reshape(1, OUT)

    loop = jnp.arange(N, dtype=jnp.int32)
    pad = NBLK_ID * BLK - E_TOT
    src = jnp.concatenate([edge_index[0], loop, jnp.zeros((pad,), jnp.int32)])
    dst = jnp.concatenate([edge_index[1], loop, jnp.full((pad,), N, jnp.int32)])
    src2d = src.reshape(NBLK_ID, BLK)
    dst2d = dst.reshape(NBLK_ID, BLK)

    zeros1 = jnp.zeros((NP, W_A1), f32)
    zeros2 = jnp.zeros((NP, W_A2), f32)

    A1, D1 = _proj1_call(xp, W1, P, Asrc, Adst)
    parts1 = _edge_pass(W_A1, 4, A1, D1, src2d, dst2d, zeros1)
    A2, S2, D2 = _mid_call(parts1, W2q, as2c, ad2c, b1c)
    parts2 = _edge_pass2(A2, S2.reshape(NP), D2.reshape(NP),
                         src2d, dst2d, zeros2)
    o = _out_call(parts2, b2r)
    return o[:N]


# speedup vs baseline: 1.0524x; 1.0524x over previous
from functools import partial

import jax
import jax.numpy as jnp
import numpy as np
from jax import lax
from jax.experimental import pallas as pl
from jax.experimental.pallas import tpu as pltpu
from jax.experimental.pallas import tpu_sc as plsc

N = 10000
NP = 10240
D_IN = 128
H1 = 8
C1 = 8
F1 = H1 * C1
OUT = 40

E_RAW = 320000
E_TOT = E_RAW + N
BLK = 128
NWORK = 32
BPW = 82
NBLK = NWORK * BPW
NBLK_ID = NBLK + 16
EP = NBLK * BLK
ROWS_PER_SUB = NP // 16

W_A1, W_A2, W_D = 80, 64, 16

_IDX_C = np.arange(F1).reshape(H1, C1).T.ravel()
_PERM = np.eye(F1, dtype=np.float32)[_IDX_C].T
_BLKDIAG_MASK = np.kron(np.eye(H1, dtype=np.float32), np.ones((C1, 1), np.float32))


def _tc_proj1(x_ref, w_ref, p_ref, asr_ref, adr_ref, a_out, d_out):
    h = jnp.dot(x_ref[...], w_ref[...], preferred_element_type=jnp.float32)
    hc = jnp.dot(h, p_ref[...], preferred_element_type=jnp.float32)
    s = jnp.dot(h, asr_ref[...], preferred_element_type=jnp.float32)
    d = jnp.dot(h, adr_ref[...], preferred_element_type=jnp.float32)
    a_out[:, 0:F1] = hc
    a_out[:, F1:F1 + H1] = s
    a_out[:, F1 + H1:W_A1] = s
    d_out[:, 0:H1] = d
    d_out[:, H1:W_D] = d


def _tc_mid(p_ref, w2_ref, as2_ref, ad2_ref, b1_ref, a_out, s_out, d_out):
    acc = p_ref[0] + p_ref[1]
    den = acc[:, F1:F1 + H1]
    div = jnp.tile(den, (1, C1)) + 1e-16
    oc = acc[:, 0:F1] / div + b1_ref[...]
    oc = jnp.maximum(oc, 0.0)
    h2 = jnp.dot(oc, w2_ref[...], preferred_element_type=jnp.float32)
    s2 = jnp.dot(h2, as2_ref[...], preferred_element_type=jnp.float32)
    d2 = jnp.dot(h2, ad2_ref[...], preferred_element_type=jnp.float32)
    rows = h2.shape[0]
    a_out[:, 0:OUT] = h2
    a_out[:, OUT:48] = jnp.zeros((rows, 48 - OUT), jnp.float32)
    s_out[...] = s2
    d_out[...] = d2


def _tc_out(p_ref, b2_ref, o_ref):
    acc = p_ref[0] + p_ref[1]
    den = acc[:, 48:49] + 1e-16
    z = acc[:, 0:OUT] / den + b2_ref[...]
    m = jnp.max(z, axis=1, keepdims=True)
    ez = jnp.exp(z - m)
    lse = jnp.log(jnp.sum(ez, axis=1, keepdims=True))
    o_ref[...] = z - m - lse


ROWB = 1280


def _proj1_call(xp, W1, P, Asrc, Adst):
    return pl.pallas_call(
        _tc_proj1,
        grid=(NP // ROWB,),
        in_specs=[
            pl.BlockSpec((ROWB, D_IN), lambda i: (i, 0)),
            pl.BlockSpec((D_IN, F1), lambda i: (0, 0)),
            pl.BlockSpec((F1, F1), lambda i: (0, 0)),
            pl.BlockSpec((F1, H1), lambda i: (0, 0)),
            pl.BlockSpec((F1, H1), lambda i: (0, 0)),
        ],
        out_specs=[
            pl.BlockSpec((ROWB, W_A1), lambda i: (i, 0)),
            pl.BlockSpec((ROWB, W_D), lambda i: (i, 0)),
        ],
        out_shape=[
            jax.ShapeDtypeStruct((NP, W_A1), jnp.float32),
            jax.ShapeDtypeStruct((NP, W_D), jnp.float32),
        ],
    )(xp, W1, P, Asrc, Adst)


def _mid_call(parts1, W2q, as2c, ad2c, b1c):
    return pl.pallas_call(
        _tc_mid,
        grid=(NP // ROWB,),
        in_specs=[
            pl.BlockSpec((2, ROWB, W_A1), lambda i: (0, i, 0)),
            pl.BlockSpec((F1, OUT), lambda i: (0, 0)),
            pl.BlockSpec((OUT, 1), lambda i: (0, 0)),
            pl.BlockSpec((OUT, 1), lambda i: (0, 0)),
            pl.BlockSpec((1, F1), lambda i: (0, 0)),
        ],
        out_specs=[
            pl.BlockSpec((ROWB, 48), lambda i: (i, 0)),
            pl.BlockSpec((ROWB, 1), lambda i: (i, 0)),
            pl.BlockSpec((ROWB, 1), lambda i: (i, 0)),
        ],
        out_shape=[
            jax.ShapeDtypeStruct((NP, 48), jnp.float32),
            jax.ShapeDtypeStruct((NP, 1), jnp.float32),
            jax.ShapeDtypeStruct((NP, 1), jnp.float32),
        ],
    )(parts1, W2q, as2c, ad2c, b1c)


def _out_call(parts2, b2r):
    return pl.pallas_call(
        _tc_out,
        grid=(NP // ROWB,),
        in_specs=[
            pl.BlockSpec((2, ROWB, W_A2), lambda i: (0, i, 0)),
            pl.BlockSpec((1, OUT), lambda i: (0, 0)),
        ],
        out_specs=pl.BlockSpec((ROWB, OUT), lambda i: (i, 0)),
        out_shape=jax.ShapeDtypeStruct((NP, OUT), jnp.float32),
    )(parts2, b2r)


def _edge_pass(width, msg_regs, a_tab, d_tab, sidx2d, didx2d, zeros_tab):
    @partial(
        pl.kernel,
        out_type=jax.ShapeDtypeStruct((2, NP, width), jnp.float32),
        mesh=plsc.VectorSubcoreMesh(core_axis_name="c", subcore_axis_name="s"),
        scratch_types=[
            pltpu.VMEM((BPW + 7, BLK), jnp.int32),
            pltpu.VMEM((BPW + 7, BLK), jnp.int32),
            pltpu.VMEM((BLK, width), jnp.float32),
            pltpu.VMEM((BLK, width), jnp.float32),
            pltpu.VMEM((BLK, W_D), jnp.float32),
            pltpu.VMEM((BLK, W_D), jnp.float32),
            pltpu.VMEM((BLK, width), jnp.float32),
            pltpu.VMEM((BLK, width), jnp.float32),
            pltpu.VMEM_SHARED((NP, width), jnp.float32),
            pltpu.SemaphoreType.DMA,
            pltpu.SemaphoreType.DMA,
            pltpu.SemaphoreType.DMA,
            pltpu.SemaphoreType.DMA,
            pltpu.SemaphoreType.DMA,
            pltpu.SemaphoreType.DMA,
        ],
        compiler_params=pltpu.CompilerParams(use_tc_tiling_on_sc=False),
    )
    def kern(a_hbm, d_hbm, s_hbm, di_hbm, z_hbm, out_hbm,
             sidx, didx, bufa0, bufa1, bufd0, bufd1, bufo0, bufo1, acc,
             semA0, semA1, semD0, semD1, semS0, semS1):
        c = lax.axis_index("c")
        s = lax.axis_index("s")
        w = c * 16 + s
        bufa = (bufa0, bufa1)
        bufd = (bufd0, bufd1)
        bufo = (bufo0, bufo1)
        semA = (semA0, semA1)
        semD = (semD0, semD1)
        semS = (semS0, semS1)
        pltpu.sync_copy(z_hbm.at[pl.ds(s * ROWS_PER_SUB, ROWS_PER_SUB)],
                        acc.at[pl.ds(s * ROWS_PER_SUB, ROWS_PER_SUB)])
        base = (w * BPW) // 8 * 8
        off = w * BPW - base
        pltpu.sync_copy(s_hbm.at[pl.ds(base, BPW + 7)], sidx)
        pltpu.sync_copy(di_hbm.at[pl.ds(base, BPW + 7)], didx)
        plsc.subcore_barrier()

        def gather_start(jj, b):
            pltpu.make_async_copy(a_hbm.at[sidx.at[off + jj]], bufa[b],
                                  semA[b]).start()
            pltpu.make_async_copy(d_hbm.at[didx.at[off + jj]], bufd[b],
                                  semD[b]).start()

        def gather_wait(jj, b):
            pltpu.make_async_copy(a_hbm.at[sidx.at[off + jj]], bufa[b],
                                  semA[b]).wait()
            pltpu.make_async_copy(d_hbm.at[didx.at[off + jj]], bufd[b],
                                  semD[b]).wait()

        def scatter_wait(jj, b):
            pltpu.make_async_copy(bufo[b], acc.at[didx.at[off + jj]],
                                  semS[b]).wait()

        gather_start(0, 0)

        def body(j, b):
            jj = j + b

            @pl.when(jj + 1 < BPW)
            def _():
                gather_start(jj + 1, 1 - b)

            gather_wait(jj, b)

            @pl.when(jj >= 2)
            def _():
                scatter_wait(jj - 2, b)

            @plsc.parallel_loop(0, BLK, 1, unroll=4)
            def _(r):
                sv = bufa[b][r, pl.ds(width - 16, 16)] + bufd[b][r, pl.ds(0, 16)]
                e = jnp.maximum(sv, 0.0) + 0.2 * jnp.minimum(sv, 0.0)
                ee = jnp.exp(e)
                for k in range(msg_regs):
                    bufo[b][r, pl.ds(16 * k, 16)] = (
                        bufa[b][r, pl.ds(16 * k, 16)] * ee)
                bufo[b][r, pl.ds(width - 16, 16)] = ee

            pltpu.make_async_copy(bufo[b], acc.at[didx.at[off + jj]],
                                  semS[b]).start(add=True)

        @pl.loop(0, BPW, step=2)
        def _(j):
            body(j, 0)
            body(j, 1)

        scatter_wait(BPW - 2, 0)
        scatter_wait(BPW - 1, 1)
        plsc.subcore_barrier()
        pltpu.sync_copy(acc.at[pl.ds(s * ROWS_PER_SUB, ROWS_PER_SUB)],
                        out_hbm.at[c].at[pl.ds(s * ROWS_PER_SUB, ROWS_PER_SUB)])

    return kern(a_tab, d_tab, sidx2d, didx2d, zeros_tab)


def _lane_bcast(vec, r):
    idx = jnp.full((16, 1), r, jnp.int32)
    dn = lax.GatherDimensionNumbers(offset_dims=(), collapsed_slice_dims=(0,),
                                    start_index_map=(0,))
    return lax.gather(vec, idx, dn, (1,),
                      mode=lax.GatherScatterMode.PROMISE_IN_BOUNDS)


def _edge_pass2(a_tab, asv_hbm, adv_hbm, sidx2d, didx2d, zeros_tab):
    width = W_A2

    @partial(
        pl.kernel,
        out_type=jax.ShapeDtypeStruct((2, NP, width), jnp.float32),
        mesh=plsc.VectorSubcoreMesh(core_axis_name="c", subcore_axis_name="s"),
        scratch_types=[
            pltpu.VMEM((BPW + 7, BLK), jnp.int32),
            pltpu.VMEM((BPW + 7, BLK), jnp.int32),
            pltpu.VMEM((BLK, 48), jnp.float32),
            pltpu.VMEM((BLK, 48), jnp.float32),
            pltpu.VMEM((BLK, width), jnp.float32),
            pltpu.VMEM((BLK, width), jnp.float32),
            pltpu.VMEM((NP,), jnp.float32),
            pltpu.VMEM((NP,), jnp.float32),
            pltpu.VMEM((16,), jnp.float32),
            pltpu.VMEM_SHARED((NP, width), jnp.float32),
            pltpu.SemaphoreType.DMA,
            pltpu.SemaphoreType.DMA,
            pltpu.SemaphoreType.DMA,
            pltpu.SemaphoreType.DMA,
        ],
        compiler_params=pltpu.CompilerParams(use_tc_tiling_on_sc=False,
                                             needs_layout_passes=False),
    )
    def kern(a_hbm, as_hbm, ad_hbm, s_hbm, di_hbm, z_hbm, out_hbm,
             sidx, didx, bufa0, bufa1, bufo0, bufo1, asv, adv, ebuf, acc,
             semA0, semA1, semS0, semS1):
        c = lax.axis_index("c")
        s = lax.axis_index("s")
        w = c * 16 + s
        bufa = (bufa0, bufa1)
        bufo = (bufo0, bufo1)
        semA = (semA0, semA1)
        semS = (semS0, semS1)
        pltpu.sync_copy(z_hbm.at[pl.ds(s * ROWS_PER_SUB, ROWS_PER_SUB)],
                        acc.at[pl.ds(s * ROWS_PER_SUB, ROWS_PER_SUB)])
        base = (w * BPW) // 8 * 8
        off = w * BPW - base
        pltpu.sync_copy(s_hbm.at[pl.ds(base, BPW + 7)], sidx)
        pltpu.sync_copy(di_hbm.at[pl.ds(base, BPW + 7)], didx)
        pltpu.sync_copy(as_hbm, asv)
        pltpu.sync_copy(ad_hbm, adv)
        plsc.subcore_barrier()

        def gather_start(jj, b):
            pltpu.make_async_copy(a_hbm.at[sidx.at[off + jj]], bufa[b],
                                  semA[b]).start()

        def gather_wait(jj, b):
            pltpu.make_async_copy(a_hbm.at[sidx.at[off + jj]], bufa[b],
                                  semA[b]).wait()

        def scatter_wait(jj, b):
            pltpu.make_async_copy(bufo[b], acc.at[didx.at[off + jj]],
                                  semS[b]).wait()

        gather_start(0, 0)

        def body(j, b):
            jj = j + b

            @pl.when(jj + 1 < BPW)
            def _():
                gather_start(jj + 1, 1 - b)

            gather_wait(jj, b)

            @pl.when(jj >= 2)
            def _():
                scatter_wait(jj - 2, b)

            @plsc.parallel_loop(0, BLK // 16, 1, unroll=2)
            def _(g):
                g16 = pl.multiple_of(g * 16, 16)
                srcv = sidx[off + jj, pl.ds(g16, 16)]
                dstv = didx[off + jj, pl.ds(g16, 16)]
                sv = (plsc.load_gather(asv, [srcv])
                      + plsc.load_gather(adv, [dstv]))
                e = jnp.maximum(sv, 0.0) + 0.2 * jnp.minimum(sv, 0.0)
                ee = jnp.exp(e)
                for r in range(16):
                    row = g16 + r
                    bc = _lane_bcast(ee, r)
                    for k in range(3):
                        bufo[b][row, pl.ds(16 * k, 16)] = (
                            bufa[b][row, pl.ds(16 * k, 16)] * bc)
                    bufo[b][row, pl.ds(48, 16)] = bc

            pltpu.make_async_copy(bufo[b], acc.at[didx.at[off + jj]],
                                  semS[b]).start(add=True)

        @pl.loop(0, BPW, step=2)
        def _(j):
            body(j, 0)
            body(j, 1)

        scatter_wait(BPW - 2, 0)
        scatter_wait(BPW - 1, 1)
        plsc.subcore_barrier()
        pltpu.sync_copy(acc.at[pl.ds(s * ROWS_PER_SUB, ROWS_PER_SUB)],
                        out_hbm.at[c].at[pl.ds(s * ROWS_PER_SUB, ROWS_PER_SUB)])

    return kern(a_tab, asv_hbm, adv_hbm, sidx2d, didx2d, zeros_tab)


@jax.jit
def kernel(x, edge_index, W1, a_src1, a_dst1, b1, W2, a_src2, a_dst2, b2):
    f32 = jnp.float32
    xp = jnp.concatenate([x, jnp.zeros((NP - N, D_IN), f32)], axis=0)

    P = jnp.asarray(_PERM)
    mask = jnp.asarray(_BLKDIAG_MASK)
    Asrc = mask * a_src1.reshape(-1, 1)
    Adst = mask * a_dst1.reshape(-1, 1)
    W2q = W2[_IDX_C, :]
    b1c = b1[_IDX_C].reshape(1, F1)
    as2c = a_src2.reshape(OUT, 1)
    ad2c = a_dst2.reshape(OUT, 1)
    b2r = b2.reshape(1, OUT)

    loop = jnp.arange(N, dtype=jnp.int32)
    pad = NBLK_ID * BLK - E_TOT
    src = jnp.concatenate([edge_index[0], loop, jnp.zeros((pad,), jnp.int32)])
    dst = jnp.concatenate([edge_index[1], loop, jnp.full((pad,), N, jnp.int32)])
    src2d = src.reshape(NBLK_ID, BLK)
    dst2d = dst.reshape(NBLK_ID, BLK)

    zeros1 = jnp.zeros((NP, W_A1), f32)
    zeros2 = jnp.zeros((NP, W_A2), f32)

    A1, D1 = _proj1_call(xp, W1, P, Asrc, Adst)
    parts1 = _edge_pass(W_A1, 4, A1, D1, src2d, dst2d, zeros1)
    A2, S2, D2 = _mid_call(parts1, W2q, as2c, ad2c, b1c)
    parts2 = _edge_pass2(A2, S2.reshape(NP), D2.reshape(NP),
                         src2d, dst2d, zeros2)
    o = _out_call(parts2, b2r)
    return o[:N]


# speedup vs baseline: 1.0897x; 1.0355x over previous
from functools import partial

import jax
import jax.numpy as jnp
import numpy as np
from jax import lax
from jax.experimental import pallas as pl
from jax.experimental.pallas import tpu as pltpu
from jax.experimental.pallas import tpu_sc as plsc

N = 10000
NP = 10240
D_IN = 128
H1 = 8
C1 = 8
F1 = H1 * C1
OUT = 40

E_RAW = 320000
E_TOT = E_RAW + N
BLK = 128
NWORK = 32
BPW = 82
NBLK = NWORK * BPW
NBLK_ID = NBLK + 16
EP = NBLK * BLK
ROWS_PER_SUB = NP // 16

W_A1, W_A2, W_D = 80, 64, 16

_IDX_C = np.arange(F1).reshape(H1, C1).T.ravel()
_PERM = np.eye(F1, dtype=np.float32)[_IDX_C].T
_BLKDIAG_MASK = np.kron(np.eye(H1, dtype=np.float32), np.ones((C1, 1), np.float32))


def _tc_proj1(x_ref, w_ref, p_ref, asr_ref, adr_ref, a_out, d_out):
    h = jnp.dot(x_ref[...], w_ref[...], preferred_element_type=jnp.float32)
    hc = jnp.dot(h, p_ref[...], preferred_element_type=jnp.float32)
    s = jnp.dot(h, asr_ref[...], preferred_element_type=jnp.float32)
    d = jnp.dot(h, adr_ref[...], preferred_element_type=jnp.float32)
    a_out[:, 0:F1] = hc
    a_out[:, F1:F1 + H1] = s
    a_out[:, F1 + H1:W_A1] = s
    d_out[:, 0:H1] = d
    d_out[:, H1:W_D] = d


def _tc_mid(p_ref, w2_ref, as2_ref, ad2_ref, b1_ref, a_out, d_out):
    acc = p_ref[0] + p_ref[1]
    den = acc[:, F1:F1 + H1]
    div = jnp.tile(den, (1, C1)) + 1e-16
    oc = acc[:, 0:F1] / div + b1_ref[...]
    oc = jnp.maximum(oc, 0.0)
    h2 = jnp.dot(oc, w2_ref[...], preferred_element_type=jnp.float32)
    s2 = jnp.dot(h2, as2_ref[...], preferred_element_type=jnp.float32)
    d2 = jnp.dot(h2, ad2_ref[...], preferred_element_type=jnp.float32)
    rows = h2.shape[0]
    a_out[:, 0:OUT] = h2
    a_out[:, OUT:48] = jnp.zeros((rows, 48 - OUT), jnp.float32)
    a_out[:, 48:W_A2] = jnp.broadcast_to(s2, (rows, 16))
    d_out[:, 0:W_D] = jnp.broadcast_to(d2, (rows, 16))


def _tc_out(p_ref, b2_ref, o_ref):
    acc = p_ref[0] + p_ref[1]
    den = acc[:, 48:49] + 1e-16
    z = acc[:, 0:OUT] / den + b2_ref[...]
    m = jnp.max(z, axis=1, keepdims=True)
    ez = jnp.exp(z - m)
    lse = jnp.log(jnp.sum(ez, axis=1, keepdims=True))
    o_ref[...] = z - m - lse


ROWB = 1280


def _proj1_call(xp, W1, P, Asrc, Adst):
    return pl.pallas_call(
        _tc_proj1,
        grid=(NP // ROWB,),
        in_specs=[
            pl.BlockSpec((ROWB, D_IN), lambda i: (i, 0)),
            pl.BlockSpec((D_IN, F1), lambda i: (0, 0)),
            pl.BlockSpec((F1, F1), lambda i: (0, 0)),
            pl.BlockSpec((F1, H1), lambda i: (0, 0)),
            pl.BlockSpec((F1, H1), lambda i: (0, 0)),
        ],
        out_specs=[
            pl.BlockSpec((ROWB, W_A1), lambda i: (i, 0)),
            pl.BlockSpec((ROWB, W_D), lambda i: (i, 0)),
        ],
        out_shape=[
            jax.ShapeDtypeStruct((NP, W_A1), jnp.float32),
            jax.ShapeDtypeStruct((NP, W_D), jnp.float32),
        ],
    )(xp, W1, P, Asrc, Adst)


def _mid_call(parts1, W2q, as2c, ad2c, b1c):
    return pl.pallas_call(
        _tc_mid,
        grid=(NP // ROWB,),
        in_specs=[
            pl.BlockSpec((2, ROWB, W_A1), lambda i: (0, i, 0)),
            pl.BlockSpec((F1, OUT), lambda i: (0, 0)),
            pl.BlockSpec((OUT, 1), lambda i: (0, 0)),
            pl.BlockSpec((OUT, 1), lambda i: (0, 0)),
            pl.BlockSpec((1, F1), lambda i: (0, 0)),
        ],
        out_specs=[
            pl.BlockSpec((ROWB, W_A2), lambda i: (i, 0)),
            pl.BlockSpec((ROWB, W_D), lambda i: (i, 0)),
        ],
        out_shape=[
            jax.ShapeDtypeStruct((NP, W_A2), jnp.float32),
            jax.ShapeDtypeStruct((NP, W_D), jnp.float32),
        ],
    )(parts1, W2q, as2c, ad2c, b1c)


def _out_call(parts2, b2r):
    return pl.pallas_call(
        _tc_out,
        grid=(NP // ROWB,),
        in_specs=[
            pl.BlockSpec((2, ROWB, W_A2), lambda i: (0, i, 0)),
            pl.BlockSpec((1, OUT), lambda i: (0, 0)),
        ],
        out_specs=pl.BlockSpec((ROWB, OUT), lambda i: (i, 0)),
        out_shape=jax.ShapeDtypeStruct((NP, OUT), jnp.float32),
    )(parts2, b2r)


def _edge_pass(width, msg_regs, a_tab, d_tab, sidx2d, didx2d, zeros_tab):
    @partial(
        pl.kernel,
        out_type=jax.ShapeDtypeStruct((2, NP, width), jnp.float32),
        mesh=plsc.VectorSubcoreMesh(core_axis_name="c", subcore_axis_name="s"),
        scratch_types=[
            pltpu.VMEM((BPW + 7, BLK), jnp.int32),
            pltpu.VMEM((BPW + 7, BLK), jnp.int32),
            pltpu.VMEM((BLK, width), jnp.float32),
            pltpu.VMEM((BLK, width), jnp.float32),
            pltpu.VMEM((BLK, W_D), jnp.float32),
            pltpu.VMEM((BLK, W_D), jnp.float32),
            pltpu.VMEM((BLK, width), jnp.float32),
            pltpu.VMEM((BLK, width), jnp.float32),
            pltpu.VMEM_SHARED((NP, width), jnp.float32),
            pltpu.SemaphoreType.DMA,
            pltpu.SemaphoreType.DMA,
            pltpu.SemaphoreType.DMA,
            pltpu.SemaphoreType.DMA,
            pltpu.SemaphoreType.DMA,
            pltpu.SemaphoreType.DMA,
        ],
        compiler_params=pltpu.CompilerParams(use_tc_tiling_on_sc=False),
    )
    def kern(a_hbm, d_hbm, s_hbm, di_hbm, z_hbm, out_hbm,
             sidx, didx, bufa0, bufa1, bufd0, bufd1, bufo0, bufo1, acc,
             semA0, semA1, semD0, semD1, semS0, semS1):
        c = lax.axis_index("c")
        s = lax.axis_index("s")
        w = c * 16 + s
        bufa = (bufa0, bufa1)
        bufd = (bufd0, bufd1)
        bufo = (bufo0, bufo1)
        semA = (semA0, semA1)
        semD = (semD0, semD1)
        semS = (semS0, semS1)
        pltpu.sync_copy(z_hbm.at[pl.ds(s * ROWS_PER_SUB, ROWS_PER_SUB)],
                        acc.at[pl.ds(s * ROWS_PER_SUB, ROWS_PER_SUB)])
        base = (w * BPW) // 8 * 8
        off = w * BPW - base
        pltpu.sync_copy(s_hbm.at[pl.ds(base, BPW + 7)], sidx)
        pltpu.sync_copy(di_hbm.at[pl.ds(base, BPW + 7)], didx)
        plsc.subcore_barrier()

        def gather_start(jj, b):
            pltpu.make_async_copy(a_hbm.at[sidx.at[off + jj]], bufa[b],
                                  semA[b]).start()
            pltpu.make_async_copy(d_hbm.at[didx.at[off + jj]], bufd[b],
                                  semD[b]).start()

        def gather_wait(jj, b):
            pltpu.make_async_copy(a_hbm.at[sidx.at[off + jj]], bufa[b],
                                  semA[b]).wait()
            pltpu.make_async_copy(d_hbm.at[didx.at[off + jj]], bufd[b],
                                  semD[b]).wait()

        def scatter_wait(jj, b):
            pltpu.make_async_copy(bufo[b], acc.at[didx.at[off + jj]],
                                  semS[b]).wait()

        gather_start(0, 0)

        def body(j, b):
            jj = j + b

            @pl.when(jj + 1 < BPW)
            def _():
                gather_start(jj + 1, 1 - b)

            gather_wait(jj, b)

            @pl.when(jj >= 2)
            def _():
                scatter_wait(jj - 2, b)

            @plsc.parallel_loop(0, BLK, 1, unroll=4)
            def _(r):
                sv = bufa[b][r, pl.ds(width - 16, 16)] + bufd[b][r, pl.ds(0, 16)]
                e = jnp.maximum(sv, 0.0) + 0.2 * jnp.minimum(sv, 0.0)
                ee = jnp.exp(e)
                for k in range(msg_regs):
                    bufo[b][r, pl.ds(16 * k, 16)] = (
                        bufa[b][r, pl.ds(16 * k, 16)] * ee)
                bufo[b][r, pl.ds(width - 16, 16)] = ee

            pltpu.make_async_copy(bufo[b], acc.at[didx.at[off + jj]],
                                  semS[b]).start(add=True)

        @pl.loop(0, BPW, step=2)
        def _(j):
            body(j, 0)
            body(j, 1)

        scatter_wait(BPW - 2, 0)
        scatter_wait(BPW - 1, 1)
        plsc.subcore_barrier()
        pltpu.sync_copy(acc.at[pl.ds(s * ROWS_PER_SUB, ROWS_PER_SUB)],
                        out_hbm.at[c].at[pl.ds(s * ROWS_PER_SUB, ROWS_PER_SUB)])

    return kern(a_tab, d_tab, sidx2d, didx2d, zeros_tab)


@jax.jit
def kernel(x, edge_index, W1, a_src1, a_dst1, b1, W2, a_src2, a_dst2, b2):
    f32 = jnp.float32
    xp = jnp.concatenate([x, jnp.zeros((NP - N, D_IN), f32)], axis=0)

    P = jnp.asarray(_PERM)
    mask = jnp.asarray(_BLKDIAG_MASK)
    Asrc = mask * a_src1.reshape(-1, 1)
    Adst = mask * a_dst1.reshape(-1, 1)
    W2q = W2[_IDX_C, :]
    b1c = b1[_IDX_C].reshape(1, F1)
    as2c = a_src2.reshape(OUT, 1)
    ad2c = a_dst2.reshape(OUT, 1)
    b2r = b2.reshape(1, OUT)

    loop = jnp.arange(N, dtype=jnp.int32)
    pad = NBLK_ID * BLK - E_TOT
    src = jnp.concatenate([edge_index[0], loop, jnp.zeros((pad,), jnp.int32)])
    dst = jnp.concatenate([edge_index[1], loop, jnp.full((pad,), N, jnp.int32)])
    src2d = src.reshape(NBLK_ID, BLK)
    dst2d = dst.reshape(NBLK_ID, BLK)

    zeros1 = jnp.zeros((NP, W_A1), f32)
    zeros2 = jnp.zeros((NP, W_A2), f32)

    A1, D1 = _proj1_call(xp, W1, P, Asrc, Adst)
    parts1 = _edge_pass(W_A1, 4, A1, D1, src2d, dst2d, zeros1)
    A2, D2 = _mid_call(parts1, W2q, as2c, ad2c, b1c)
    parts2 = _edge_pass(W_A2, 3, A2, D2, src2d, dst2d, zeros2)
    o = _out_call(parts2, b2r)
    return o[:N]
